# Initial kernel scaffold; baseline (speedup 1.0000x reference)
import functools

import jax
import jax.numpy as jnp
from jax import lax
from jax.experimental import pallas as pl
from jax.experimental.pallas import tpu as pltpu
from jax.experimental.pallas import tpu_sc as plsc

N = 10000
E = 320000
K = 27
CH = 32
HID = 24
OUT = 8

NPAD = 10240
NB = 128
GRID = NPAD // NB
NSUB = 16
ECH = 128
NCHUNK = 160
EPAD = NSUB * NCHUNK * ECH
ZROWS = 160
RPS = NPAD // NSUB

f32 = jnp.float32


def _sc_body(y_hbm, gidx_hbm, dst_hbm, out_hbm,
             gidx_v, dst_v, zbuf, gbuf, acc, sem, *, fh):
    c = lax.axis_index("c")
    s = lax.axis_index("s")

    def zrow(i, carry):
        for j in range(fh // 16):
            zbuf[i, pl.ds(j * 16, 16)] = jnp.zeros((16,), f32)
        return carry
    lax.fori_loop(0, ZROWS, zrow, 0)
    for r in range(RPS // ZROWS):
        pltpu.sync_copy(zbuf, acc.at[pl.ds(s * RPS + r * ZROWS, ZROWS)])
    plsc.subcore_barrier()

    pltpu.sync_copy(gidx_hbm.at[s], gidx_v)
    pltpu.sync_copy(dst_hbm.at[s], dst_v)

    yc = y_hbm.at[c]

    def chunk(j, carry):
        pltpu.async_copy(yc.at[gidx_v.at[j]], gbuf, sem).wait()
        pltpu.sync_copy(gbuf, acc.at[dst_v.at[j]], add=True)
        return carry
    lax.fori_loop(0, NCHUNK, chunk, 0)
    plsc.subcore_barrier()

    pltpu.sync_copy(acc.at[pl.ds(s * RPS, RPS)],
                    out_hbm.at[c, pl.ds(s * RPS, RPS)])


def _sc_agg(y2, gidx, dst, fh):
    mesh = plsc.VectorSubcoreMesh(core_axis_name="c", subcore_axis_name="s")
    fn = pl.kernel(
        functools.partial(_sc_body, fh=fh),
        out_type=jax.ShapeDtypeStruct((2, NPAD, fh), f32),
        mesh=mesh,
        scratch_types=[
            pltpu.VMEM((NCHUNK, ECH), jnp.int32),
            pltpu.VMEM((NCHUNK, ECH), jnp.int32),
            pltpu.VMEM((ZROWS, fh), f32),
            pltpu.VMEM((ECH, fh), f32),
            pltpu.VMEM_SHARED((NPAD, fh), f32),
            pltpu.SemaphoreType.DMA,
        ],
    )
    return fn(y2, gidx, dst)


def _yspec(fh):
    return pl.BlockSpec((2, NB, K * fh), lambda i: (0, i, 0))


def _aspec(fh):
    return pl.BlockSpec((2, NB, fh), lambda i: (0, i, 0))


def _nspec(w):
    return pl.BlockSpec((NB, w), lambda i: (i, 0))


def _fspec(shape):
    return pl.BlockSpec(shape, lambda i: (0,) * len(shape))


def _cat(a_ref):
    return jnp.concatenate([a_ref[0], a_ref[1]], axis=-1)


def _t1(x_ref, w_ref, y_ref):
    x = x_ref[...]
    y_ref[0] = jnp.dot(x, w_ref[0], preferred_element_type=f32)
    y_ref[1] = jnp.dot(x, w_ref[1], preferred_element_type=f32)


def _t_lin(a_ref, b_ref, w_ref, y_ref):
    h = jnp.maximum(_cat(a_ref) + b_ref[0], 0.0)
    y_ref[0] = jnp.dot(h, w_ref[0], preferred_element_type=f32)
    y_ref[1] = jnp.dot(h, w_ref[1], preferred_element_type=f32)


def _t4(a1_ref, a3_ref, b0_ref, b2_ref, w_ref, y_ref):
    h = jnp.maximum(_cat(a1_ref) + b0_ref[0], 0.0)
    r2 = _cat(a3_ref) + b2_ref[0]
    h2 = jnp.maximum(h + r2, 0.0)
    y_ref[0] = jnp.dot(h2, w_ref[0], preferred_element_type=f32)
    y_ref[1] = jnp.dot(h2, w_ref[1], preferred_element_type=f32)


def _t5(x_ref, w_ref, y_ref):
    x = x_ref[...]
    y_ref[0] = jnp.dot(x, w_ref[0], preferred_element_type=f32)
    y_ref[1] = jnp.dot(x, w_ref[1], preferred_element_type=f32)


def _halves(parts, nbr):
    yt = jnp.stack(parts, axis=2).reshape(NB, K, nbr * CH)
    fh = nbr * CH // 2
    return (yt[:, :, :fh].reshape(NB, K * fh),
            yt[:, :, fh:].reshape(NB, K * fh))


def _t_branch(a_ref, b_ref, w_ref, y_ref):
    a = _cat(a_ref).reshape(NB, OUT - 1, CH)
    parts = []
    for j in range(OUT - 1):
        h = jnp.maximum(a[:, j] + b_ref[j], 0.0)
        parts.append(jnp.dot(h, w_ref[j], preferred_element_type=f32)
                     .reshape(NB, K, CH))
    y0, y1 = _halves(parts, OUT - 1)
    y_ref[0] = y0
    y_ref[1] = y1


def _t8(a5_ref, a7_ref, b0_ref, b2_ref, w_ref, y_ref):
    a5 = _cat(a5_ref).reshape(NB, OUT - 1, CH)
    a7 = _cat(a7_ref).reshape(NB, OUT - 1, CH)
    parts = []
    for j in range(OUT - 1):
        ho = jnp.maximum(a5[:, j] + b0_ref[j], 0.0)
        h2 = jnp.maximum(ho + a7[:, j] + b2_ref[j], 0.0)
        parts.append(jnp.dot(h2, w_ref[j], preferred_element_type=f32)
                     .reshape(NB, K, CH))
    y0, y1 = _halves(parts, OUT - 1)
    y_ref[0] = y0
    y_ref[1] = y1


def _t9(a4_ref, b3_ref, a8_ref, otb3_ref, w_ref, y_ref):
    xg = _cat(a4_ref) + b3_ref[0]
    gg = _cat(a8_ref).reshape(NB, OUT - 1, CH)
    parts = []
    for oi in range(OUT):
        if oi == 0:
            inp = xg
        else:
            inp = xg + gg[:, oi - 1] + otb3_ref[oi - 1]
        parts.append(jnp.dot(inp, w_ref[oi], preferred_element_type=f32)
                     .reshape(NB, K, CH))
    y0, y1 = _halves(parts, OUT)
    y_ref[0] = y0
    y_ref[1] = y1


def _t10(apr_ref, prb_ref, w1_ref, b1_ref, w2_ref, b2_ref, out_ref):
    a = _cat(apr_ref).reshape(NB, OUT, CH)
    cols = []
    for oi in range(OUT):
        t = a[:, oi] + prb_ref[oi]
        y1 = jnp.maximum(
            jnp.dot(t, w1_ref[oi], preferred_element_type=f32) + b1_ref[oi],
            0.0)
        z = jnp.dot(y1, w2_ref[oi], preferred_element_type=f32) + b2_ref[oi]
        cols.append(1.0 / (1.0 + jnp.exp(-z)))
    out_ref[...] = jnp.concatenate(cols, axis=1)


def _call(body, in_arrs, in_specs, out_shape, out_spec):
    return pl.pallas_call(
        body, grid=(GRID,), in_specs=in_specs,
        out_specs=out_spec,
        out_shape=jax.ShapeDtypeStruct(out_shape, f32),
    )(*in_arrs)


def kernel(x_low, x_occ, edge_index, edge_offset,
           bi_W0, bi_b0, bi_W1, bi_b1, bi_W2, bi_b2, bi_W3, bi_b3,
           pr_W, pr_b, ml_W1, ml_b1, ml_W2, ml_b2,
           ot_W0, ot_b0, ot_W1, ot_b1, ot_W2, ot_b2, ot_W3, ot_b3):
    src = edge_index[0]
    dstv = edge_index[1]
    off = edge_offset

    gidx = src * K + off
    pad = EPAD - E
    gidx_t = jnp.concatenate(
        [gidx, jnp.zeros((pad,), jnp.int32)]).reshape(NSUB, NCHUNK, ECH)
    dst_t = jnp.concatenate(
        [dstv, jnp.full((pad,), N, jnp.int32)]).reshape(NSUB, NCHUNK, ECH)

    xl = jnp.pad(x_low, ((0, NPAD - N), (0, 0)))
    x7 = jnp.pad(x_occ[:, :OUT - 1], ((0, NPAD - N), (0, 0)))

    def wsplit(W):
        Kk, Cin, Ft = W.shape
        return (W.transpose(1, 0, 2).reshape(Cin, Kk, 2, Ft // 2)
                .transpose(2, 0, 1, 3).reshape(2, Cin, Kk * (Ft // 2)))

    def wbr(W):
        B = W.shape[0]
        return W.transpose(0, 2, 1, 3).reshape(B, CH, K * CH)

    w_bi0 = wsplit(bi_W0)
    w_bi1 = wsplit(bi_W1)
    w_bi2 = wsplit(bi_W2)
    w_bi3 = wsplit(bi_W3)

    cmask = (jnp.arange(OUT - 1)[None, :] <= jnp.arange(OUT - 1)[:, None])
    W0m = ot_W0 * cmask[:, None, :, None].astype(f32)
    Wcat1 = W0m.transpose(1, 2, 0, 3).reshape(K, OUT - 1, (OUT - 1) * CH)
    w_ot0 = wsplit(Wcat1)

    w_ot1 = wbr(ot_W1)
    w_ot2 = wbr(ot_W2)
    w_ot3 = wbr(ot_W3)
    w_pr = wbr(pr_W)

    b_bi0 = bi_b0.reshape(1, CH)
    b_bi1 = bi_b1.reshape(1, CH)
    b_bi2 = bi_b2.reshape(1, CH)
    b_bi3 = bi_b3.reshape(1, CH)

    def agg(y, fh):
        return _sc_agg(y.reshape(2, NPAD * K, fh), gidx_t, dst_t, fh)

    y = _call(_t1, (xl, w_bi0), [_nspec(1), _fspec(w_bi0.shape)],
              (2, NPAD, K * 16), _yspec(16))
    a1 = agg(y, 16)
    y = _call(_t_lin, (a1, b_bi0, w_bi1),
              [_aspec(16), _fspec((1, CH)), _fspec(w_bi1.shape)],
              (2, NPAD, K * 16), _yspec(16))
    a2 = agg(y, 16)
    y = _call(_t_lin, (a2, b_bi1, w_bi2),
              [_aspec(16), _fspec((1, CH)), _fspec(w_bi2.shape)],
              (2, NPAD, K * 16), _yspec(16))
    a3 = agg(y, 16)
    y = _call(_t4, (a1, a3, b_bi0, b_bi2, w_bi3),
              [_aspec(16), _aspec(16), _fspec((1, CH)), _fspec((1, CH)),
               _fspec(w_bi3.shape)],
              (2, NPAD, K * 16), _yspec(16))
    a4 = agg(y, 16)

    fh7 = (OUT - 1) * CH // 2
    y = _call(_t5, (x7, w_ot0), [_nspec(OUT - 1), _fspec(w_ot0.shape)],
              (2, NPAD, K * fh7), _yspec(fh7))
    a5 = agg(y, fh7)
    y = _call(_t_branch, (a5, ot_b0, w_ot1),
              [_aspec(fh7), _fspec(ot_b0.shape), _fspec(w_ot1.shape)],
              (2, NPAD, K * fh7), _yspec(fh7))
    a6 = agg(y, fh7)
    y = _call(_t_branch, (a6, ot_b1, w_ot2),
              [_aspec(fh7), _fspec(ot_b1.shape), _fspec(w_ot2.shape)],
              (2, NPAD, K * fh7), _yspec(fh7))
    a7 = agg(y, fh7)
    y = _call(_t8, (a5, a7, ot_b0, ot_b2, w_ot3),
              [_aspec(fh7), _aspec(fh7), _fspec(ot_b0.shape),
               _fspec(ot_b2.shape), _fspec(w_ot3.shape)],
              (2, NPAD, K * fh7), _yspec(fh7))
    a8 = agg(y, fh7)

    fh8 = OUT * CH // 2
    y = _call(_t9, (a4, b_bi3, a8, ot_b3, w_pr),
              [_aspec(16), _fspec((1, CH)), _aspec(fh7), _fspec(ot_b3.shape),
               _fspec(w_pr.shape)],
              (2, NPAD, K * fh8), _yspec(fh8))
    apr = agg(y, fh8)

    out = _call(_t10, (apr, pr_b, ml_W1, ml_b1, ml_W2, ml_b2),
                [_aspec(fh8), _fspec(pr_b.shape), _fspec(ml_W1.shape),
                 _fspec(ml_b1.shape), _fspec(ml_W2.shape),
                 _fspec(ml_b2.shape)],
                (NPAD, OUT), _nspec(OUT))
    return out[:N]


# speedup vs baseline: 6.6862x; 6.6862x over previous
import functools

import jax
import jax.numpy as jnp
from jax import lax
from jax.experimental import pallas as pl
from jax.experimental.pallas import tpu as pltpu
from jax.experimental.pallas import tpu_sc as plsc

N = 10000
E = 320000
K = 27
CH = 32
HID = 24
OUT = 8

NPAD = 10240
NB = 128
GRID = NPAD // NB
NSUB = 16
ECH = 128
NCHUNK = 160
IB = 16
NIB = NCHUNK // IB
EPAD = NSUB * NCHUNK * ECH
ZROWS = 40
RPS = NPAD // NSUB
FH = 128

f32 = jnp.float32


def _sc_body(y_hbm, gidx_hbm, dst_hbm, out_hbm,
             gidx_v, dst_v, zbuf, gbuf, acc, sem):
    c = lax.axis_index("c")
    s = lax.axis_index("s")

    def zrow(i, carry):
        for j in range(FH // 16):
            zbuf[i, pl.ds(j * 16, 16)] = jnp.zeros((16,), f32)
        return carry
    lax.fori_loop(0, ZROWS, zrow, 0)
    for r in range(RPS // ZROWS):
        pltpu.sync_copy(zbuf, acc.at[pl.ds(s * RPS + r * ZROWS, ZROWS)])
    plsc.subcore_barrier()

    yc = y_hbm.at[c]

    def blk(ib, carry):
        pltpu.sync_copy(gidx_hbm.at[s, pl.ds(ib * IB, IB)], gidx_v)
        pltpu.sync_copy(dst_hbm.at[s, pl.ds(ib * IB, IB)], dst_v)

        def chunk(j, carry2):
            pltpu.async_copy(yc.at[gidx_v.at[j]], gbuf, sem).wait()
            pltpu.sync_copy(gbuf, acc.at[dst_v.at[j]], add=True)
            return carry2
        lax.fori_loop(0, IB, chunk, 0)
        return carry
    lax.fori_loop(0, NIB, blk, 0)
    plsc.subcore_barrier()

    pltpu.sync_copy(acc.at[pl.ds(s * RPS, RPS)],
                    out_hbm.at[c, pl.ds(s * RPS, RPS)])


def _sc_agg(y, gidx, dst):
    mesh = plsc.VectorSubcoreMesh(core_axis_name="c", subcore_axis_name="s")
    fn = pl.kernel(
        _sc_body,
        out_type=jax.ShapeDtypeStruct((2, NPAD, FH), f32),
        mesh=mesh,
        scratch_types=[
            pltpu.VMEM((IB, ECH), jnp.int32),
            pltpu.VMEM((IB, ECH), jnp.int32),
            pltpu.VMEM((ZROWS, FH), f32),
            pltpu.VMEM((ECH, FH), f32),
            pltpu.VMEM_SHARED((NPAD, FH), f32),
            pltpu.SemaphoreType.DMA,
        ],
    )
    return fn(y.reshape(2, K * NPAD, FH), gidx, dst)


def _yspec():
    return pl.BlockSpec((2, K, NB, FH), lambda i: (0, 0, i, 0))


def _aspec():
    return pl.BlockSpec((2, NB, FH), lambda i: (0, i, 0))


def _nspec(w):
    return pl.BlockSpec((NB, w), lambda i: (i, 0))


def _fspec(shape):
    return pl.BlockSpec(shape, lambda i: (0,) * len(shape))


def _cat(a_ref):
    return jnp.concatenate([a_ref[0], a_ref[1]], axis=-1)


def _emit(h8, w_ref, y_ref):
    cin = w_ref.shape[2]
    for k in range(K):
        parts = [
            jnp.dot(h8[:, b * cin:(b + 1) * cin], w_ref[b, k],
                    preferred_element_type=f32)
            for b in range(OUT)
        ]
        row = jnp.concatenate(parts, axis=1)
        y_ref[0, k] = row[:, :FH]
        y_ref[1, k] = row[:, FH:]


def _t_first(x_ref, w_ref, y_ref):
    x = x_ref[...]
    h8 = jnp.concatenate([x] * OUT, axis=1)
    _emit(h8, w_ref, y_ref)


def _t_lin(a_ref, b0_ref, w_ref, y_ref):
    a = _cat(a_ref)
    h8 = jnp.maximum(a + b0_ref[0], 0.0)
    _emit(h8, w_ref, y_ref)


def _t_res(a1_ref, a3_ref, b0_ref, b2_ref, w_ref, y_ref):
    h = jnp.maximum(_cat(a1_ref) + b0_ref[0], 0.0)
    r2 = _cat(a3_ref) + b2_ref[0]
    h8 = jnp.maximum(h + r2, 0.0)
    _emit(h8, w_ref, y_ref)


def _t_pr(a4_ref, b3_ref, w_ref, y_ref):
    a = _cat(a_ref=a4_ref)
    g = a + b3_ref[0]
    xg = g[:, :CH]
    parts = [xg] + [xg + g[:, oi * CH:(oi + 1) * CH] for oi in range(1, OUT)]
    h8 = jnp.concatenate(parts, axis=1)
    _emit(h8, w_ref, y_ref)


def _t_head(apr_ref, prb_ref, w1_ref, b1_ref, w2_ref, b2_ref, out_ref):
    a = _cat(apr_ref).reshape(NB, OUT, CH)
    cols = []
    for oi in range(OUT):
        t = a[:, oi] + prb_ref[oi]
        y1 = jnp.maximum(
            jnp.dot(t, w1_ref[oi], preferred_element_type=f32) + b1_ref[oi],
            0.0)
        z = jnp.dot(y1, w2_ref[oi], preferred_element_type=f32) + b2_ref[oi]
        cols.append(1.0 / (1.0 + jnp.exp(-z)))
    out_ref[...] = jnp.concatenate(cols, axis=1)


def _call(body, in_arrs, in_specs, out_shape, out_spec):
    return pl.pallas_call(
        body, grid=(GRID,), in_specs=in_specs,
        out_specs=out_spec,
        out_shape=jax.ShapeDtypeStruct(out_shape, f32),
    )(*in_arrs)


def kernel(x_low, x_occ, edge_index, edge_offset,
           bi_W0, bi_b0, bi_W1, bi_b1, bi_W2, bi_b2, bi_W3, bi_b3,
           pr_W, pr_b, ml_W1, ml_b1, ml_W2, ml_b2,
           ot_W0, ot_b0, ot_W1, ot_b1, ot_W2, ot_b2, ot_W3, ot_b3):
    src = edge_index[0]
    dstv = edge_index[1]
    off = edge_offset

    gidx = off * NPAD + src
    pad = EPAD - E
    gidx_t = jnp.concatenate(
        [gidx, jnp.zeros((pad,), jnp.int32)]).reshape(NSUB, NCHUNK, ECH)
    dst_t = jnp.concatenate(
        [dstv, jnp.full((pad,), N, jnp.int32)]).reshape(NSUB, NCHUNK, ECH)

    x8 = jnp.concatenate(
        [jnp.pad(x_low, ((0, NPAD - N), (0, 0))),
         jnp.pad(x_occ[:, :OUT - 1], ((0, NPAD - N), (0, 0)))], axis=1)

    cmask = (jnp.arange(OUT - 1)[None, :] <= jnp.arange(OUT - 1)[:, None])
    W0m = ot_W0 * cmask[:, None, :, None].astype(f32)
    wp1 = jnp.concatenate(
        [jnp.pad(bi_W0[None], ((0, 0), (0, 0), (0, OUT - 1), (0, 0))),
         jnp.pad(W0m, ((0, 0), (0, 0), (1, 0), (0, 0)))], axis=0)
    w1 = jnp.concatenate([bi_W1[None], ot_W1], axis=0)
    w2 = jnp.concatenate([bi_W2[None], ot_W2], axis=0)
    w3 = jnp.concatenate([bi_W3[None], ot_W3], axis=0)
    b0 = jnp.concatenate([bi_b0[None], ot_b0], axis=0).reshape(1, OUT * CH)
    b1 = jnp.concatenate([bi_b1[None], ot_b1], axis=0).reshape(1, OUT * CH)
    b2 = jnp.concatenate([bi_b2[None], ot_b2], axis=0).reshape(1, OUT * CH)
    b3 = jnp.concatenate([bi_b3[None], ot_b3], axis=0).reshape(1, OUT * CH)

    def agg(y):
        return _sc_agg(y, gidx_t, dst_t)

    yshape = (2, K, NPAD, FH)

    y = _call(_t_first, (x8, wp1), [_nspec(OUT), _fspec(wp1.shape)],
              yshape, _yspec())
    a1 = agg(y)
    y = _call(_t_lin, (a1, b0, w1),
              [_aspec(), _fspec(b0.shape), _fspec(w1.shape)],
              yshape, _yspec())
    a2 = agg(y)
    y = _call(_t_lin, (a2, b1, w2),
              [_aspec(), _fspec(b1.shape), _fspec(w2.shape)],
              yshape, _yspec())
    a3 = agg(y)
    y = _call(_t_res, (a1, a3, b0, b2, w3),
              [_aspec(), _aspec(), _fspec(b0.shape), _fspec(b2.shape),
               _fspec(w3.shape)],
              yshape, _yspec())
    a4 = agg(y)
    y = _call(_t_pr, (a4, b3, pr_W),
              [_aspec(), _fspec(b3.shape), _fspec(pr_W.shape)],
              yshape, _yspec())
    apr = agg(y)

    out = _call(_t_head, (apr, pr_b, ml_W1, ml_b1, ml_W2, ml_b2),
                [_aspec(), _fspec(pr_b.shape), _fspec(ml_W1.shape),
                 _fspec(ml_b1.shape), _fspec(ml_W2.shape),
                 _fspec(ml_b2.shape)],
                (NPAD, OUT), _nspec(OUT))
    return out[:N]


# speedup vs baseline: 7.5984x; 1.1364x over previous
import functools

import jax
import jax.numpy as jnp
from jax import lax
from jax.experimental import pallas as pl
from jax.experimental.pallas import tpu as pltpu
from jax.experimental.pallas import tpu_sc as plsc

N = 10000
E = 320000
K = 27
CH = 32
HID = 24
OUT = 8

NPAD = 10240
NB = 128
GRID = NPAD // NB
NSUB = 16
ECH = 128
NCHUNK = 160
IB = 16
NIB = NCHUNK // IB
EPAD = NSUB * NCHUNK * ECH
ZROWS = 16
RPS = NPAD // NSUB
FH = 128

f32 = jnp.float32


def _sc_body(y_hbm, gidx_hbm, dst_hbm, out_hbm,
             gidx_v, dst_v, zbuf, gbuf0, gbuf1, acc,
             semg0, semg1, sems0, sems1):
    c = lax.axis_index("c")
    s = lax.axis_index("s")

    def zrow(i, carry):
        for j in range(FH // 16):
            zbuf[i, pl.ds(j * 16, 16)] = jnp.zeros((16,), f32)
        return carry
    lax.fori_loop(0, ZROWS, zrow, 0)
    for r in range(RPS // ZROWS):
        pltpu.sync_copy(zbuf, acc.at[pl.ds(s * RPS + r * ZROWS, ZROWS)])
    plsc.subcore_barrier()

    yc = y_hbm.at[c]
    gb = (gbuf0, gbuf1)
    semg = (semg0, semg1)
    sems = (sems0, sems1)

    def blk(ib, carry):
        pltpu.sync_copy(gidx_hbm.at[s, pl.ds(ib * IB, IB)], gidx_v)
        pltpu.sync_copy(dst_hbm.at[s, pl.ds(ib * IB, IB)], dst_v)

        g_desc = [None, None]
        s_desc = [None, None]
        g_desc[0] = pltpu.async_copy(yc.at[gidx_v.at[0]], gb[0], semg[0])
        for j in range(IB):
            p = j % 2
            q = 1 - p
            if j + 1 < IB:
                if j >= 1:
                    s_desc[q].wait()
                g_desc[q] = pltpu.async_copy(
                    yc.at[gidx_v.at[j + 1]], gb[q], semg[q])
            g_desc[p].wait()
            s_desc[p] = pltpu.async_copy(
                gb[p], acc.at[dst_v.at[j]], sems[p], add=True)
        s_desc[0].wait()
        s_desc[1].wait()
        return carry
    lax.fori_loop(0, NIB, blk, 0)
    plsc.subcore_barrier()

    pltpu.sync_copy(acc.at[pl.ds(s * RPS, RPS)],
                    out_hbm.at[c, pl.ds(s * RPS, RPS)])


def _sc_agg(y, gidx, dst):
    mesh = plsc.VectorSubcoreMesh(core_axis_name="c", subcore_axis_name="s")
    fn = pl.kernel(
        _sc_body,
        out_type=jax.ShapeDtypeStruct((2, NPAD, FH), f32),
        mesh=mesh,
        scratch_types=[
            pltpu.VMEM((IB, ECH), jnp.int32),
            pltpu.VMEM((IB, ECH), jnp.int32),
            pltpu.VMEM((ZROWS, FH), f32),
            pltpu.VMEM((ECH, FH), f32),
            pltpu.VMEM((ECH, FH), f32),
            pltpu.VMEM_SHARED((NPAD, FH), f32),
            pltpu.SemaphoreType.DMA,
            pltpu.SemaphoreType.DMA,
            pltpu.SemaphoreType.DMA,
            pltpu.SemaphoreType.DMA,
        ],
    )
    return fn(y.reshape(2, K * NPAD, FH), gidx, dst)


def _yspec():
    return pl.BlockSpec((2, K, NB, FH), lambda i: (0, 0, i, 0))


def _aspec():
    return pl.BlockSpec((2, NB, FH), lambda i: (0, i, 0))


def _nspec(w):
    return pl.BlockSpec((NB, w), lambda i: (i, 0))


def _fspec(shape):
    return pl.BlockSpec(shape, lambda i: (0,) * len(shape))


def _cat(a_ref):
    return jnp.concatenate([a_ref[0], a_ref[1]], axis=-1)


def _emit(h8, w_ref, y_ref):
    cin = w_ref.shape[2]
    for k in range(K):
        parts = [
            jnp.dot(h8[:, b * cin:(b + 1) * cin], w_ref[b, k],
                    preferred_element_type=f32)
            for b in range(OUT)
        ]
        row = jnp.concatenate(parts, axis=1)
        y_ref[0, k] = row[:, :FH]
        y_ref[1, k] = row[:, FH:]


def _t_first(x_ref, w_ref, y_ref):
    x = x_ref[...]
    h8 = jnp.concatenate([x] * OUT, axis=1)
    _emit(h8, w_ref, y_ref)


def _t_lin(a_ref, b0_ref, w_ref, y_ref):
    a = _cat(a_ref)
    h8 = jnp.maximum(a + b0_ref[0], 0.0)
    _emit(h8, w_ref, y_ref)


def _t_res(a1_ref, a3_ref, b0_ref, b2_ref, w_ref, y_ref):
    h = jnp.maximum(_cat(a1_ref) + b0_ref[0], 0.0)
    r2 = _cat(a3_ref) + b2_ref[0]
    h8 = jnp.maximum(h + r2, 0.0)
    _emit(h8, w_ref, y_ref)


def _t_pr(a4_ref, b3_ref, w_ref, y_ref):
    a = _cat(a_ref=a4_ref)
    g = a + b3_ref[0]
    xg = g[:, :CH]
    parts = [xg] + [xg + g[:, oi * CH:(oi + 1) * CH] for oi in range(1, OUT)]
    h8 = jnp.concatenate(parts, axis=1)
    _emit(h8, w_ref, y_ref)


def _t_head(apr_ref, prb_ref, w1_ref, b1_ref, w2_ref, b2_ref, out_ref):
    a = _cat(apr_ref).reshape(NB, OUT, CH)
    cols = []
    for oi in range(OUT):
        t = a[:, oi] + prb_ref[oi]
        y1 = jnp.maximum(
            jnp.dot(t, w1_ref[oi], preferred_element_type=f32) + b1_ref[oi],
            0.0)
        z = jnp.dot(y1, w2_ref[oi], preferred_element_type=f32) + b2_ref[oi]
        cols.append(1.0 / (1.0 + jnp.exp(-z)))
    out_ref[...] = jnp.concatenate(cols, axis=1)


def _call(body, in_arrs, in_specs, out_shape, out_spec):
    return pl.pallas_call(
        body, grid=(GRID,), in_specs=in_specs,
        out_specs=out_spec,
        out_shape=jax.ShapeDtypeStruct(out_shape, f32),
    )(*in_arrs)


def kernel(x_low, x_occ, edge_index, edge_offset,
           bi_W0, bi_b0, bi_W1, bi_b1, bi_W2, bi_b2, bi_W3, bi_b3,
           pr_W, pr_b, ml_W1, ml_b1, ml_W2, ml_b2,
           ot_W0, ot_b0, ot_W1, ot_b1, ot_W2, ot_b2, ot_W3, ot_b3):
    src = edge_index[0]
    dstv = edge_index[1]
    off = edge_offset

    gidx = off * NPAD + src
    pad = EPAD - E
    gidx_t = jnp.concatenate(
        [gidx, jnp.zeros((pad,), jnp.int32)]).reshape(NSUB, NCHUNK, ECH)
    dst_t = jnp.concatenate(
        [dstv, jnp.full((pad,), N, jnp.int32)]).reshape(NSUB, NCHUNK, ECH)

    x8 = jnp.concatenate(
        [jnp.pad(x_low, ((0, NPAD - N), (0, 0))),
         jnp.pad(x_occ[:, :OUT - 1], ((0, NPAD - N), (0, 0)))], axis=1)

    cmask = (jnp.arange(OUT - 1)[None, :] <= jnp.arange(OUT - 1)[:, None])
    W0m = ot_W0 * cmask[:, None, :, None].astype(f32)
    wp1 = jnp.concatenate(
        [jnp.pad(bi_W0[None], ((0, 0), (0, 0), (0, OUT - 1), (0, 0))),
         jnp.pad(W0m, ((0, 0), (0, 0), (1, 0), (0, 0)))], axis=0)
    w1 = jnp.concatenate([bi_W1[None], ot_W1], axis=0)
    w2 = jnp.concatenate([bi_W2[None], ot_W2], axis=0)
    w3 = jnp.concatenate([bi_W3[None], ot_W3], axis=0)
    b0 = jnp.concatenate([bi_b0[None], ot_b0], axis=0).reshape(1, OUT * CH)
    b1 = jnp.concatenate([bi_b1[None], ot_b1], axis=0).reshape(1, OUT * CH)
    b2 = jnp.concatenate([bi_b2[None], ot_b2], axis=0).reshape(1, OUT * CH)
    b3 = jnp.concatenate([bi_b3[None], ot_b3], axis=0).reshape(1, OUT * CH)

    def agg(y):
        return _sc_agg(y, gidx_t, dst_t)

    yshape = (2, K, NPAD, FH)

    y = _call(_t_first, (x8, wp1), [_nspec(OUT), _fspec(wp1.shape)],
              yshape, _yspec())
    a1 = agg(y)
    y = _call(_t_lin, (a1, b0, w1),
              [_aspec(), _fspec(b0.shape), _fspec(w1.shape)],
              yshape, _yspec())
    a2 = agg(y)
    y = _call(_t_lin, (a2, b1, w2),
              [_aspec(), _fspec(b1.shape), _fspec(w2.shape)],
              yshape, _yspec())
    a3 = agg(y)
    y = _call(_t_res, (a1, a3, b0, b2, w3),
              [_aspec(), _aspec(), _fspec(b0.shape), _fspec(b2.shape),
               _fspec(w3.shape)],
              yshape, _yspec())
    a4 = agg(y)
    y = _call(_t_pr, (a4, b3, pr_W),
              [_aspec(), _fspec(b3.shape), _fspec(pr_W.shape)],
              yshape, _yspec())
    apr = agg(y)

    out = _call(_t_head, (apr, pr_b, ml_W1, ml_b1, ml_W2, ml_b2),
                [_aspec(), _fspec(pr_b.shape), _fspec(ml_W1.shape),
                 _fspec(ml_b1.shape), _fspec(ml_W2.shape),
                 _fspec(ml_b2.shape)],
                (NPAD, OUT), _nspec(OUT))
    return out[:N]


# speedup vs baseline: 9.5605x; 1.2582x over previous
import functools

import jax
import jax.numpy as jnp
from jax import lax
from jax.experimental import pallas as pl
from jax.experimental.pallas import tpu as pltpu
from jax.experimental.pallas import tpu_sc as plsc

N = 10000
E = 320000
K = 27
CH = 32
HID = 24
OUT = 8

NPAD = 10240
NB = 128
GRID = NPAD // NB
NSUB = 16
ECH = 128
NCHUNK = 160
IB = 16
NIB = NCHUNK // IB
EPAD = NSUB * NCHUNK * ECH
ZROWS = 16
RPS = NPAD // NSUB
FH = 128

f32 = jnp.float32


def _sc_body(y_hbm, gidx_hbm, dst_hbm, out_hbm,
             gidx_v, dst_v, zbuf, gbuf0, gbuf1, acc,
             semg0, semg1, sems0, sems1):
    c = lax.axis_index("c")
    s = lax.axis_index("s")

    def zrow(i, carry):
        for j in range(FH // 16):
            zbuf[i, pl.ds(j * 16, 16)] = jnp.zeros((16,), f32)
        return carry
    lax.fori_loop(0, ZROWS, zrow, 0)
    for r in range(RPS // ZROWS):
        pltpu.sync_copy(zbuf, acc.at[pl.ds(s * RPS + r * ZROWS, ZROWS)])
    plsc.subcore_barrier()

    yc = y_hbm.at[c]
    gb = (gbuf0, gbuf1)
    semg = (semg0, semg1)
    sems = (sems0, sems1)

    def blk(ib, carry):
        pltpu.sync_copy(gidx_hbm.at[s, pl.ds(ib * IB, IB)], gidx_v)
        pltpu.sync_copy(dst_hbm.at[s, pl.ds(ib * IB, IB)], dst_v)

        g_desc = [None, None]
        s_desc = [None, None]
        g_desc[0] = pltpu.async_copy(yc.at[gidx_v.at[0]], gb[0], semg[0])
        for j in range(IB):
            p = j % 2
            q = 1 - p
            if j + 1 < IB:
                if j >= 1:
                    s_desc[q].wait()
                g_desc[q] = pltpu.async_copy(
                    yc.at[gidx_v.at[j + 1]], gb[q], semg[q])
            g_desc[p].wait()
            s_desc[p] = pltpu.async_copy(
                gb[p], acc.at[dst_v.at[j]], sems[p], add=True)
        s_desc[0].wait()
        s_desc[1].wait()
        return carry
    lax.fori_loop(0, NIB, blk, 0)
    plsc.subcore_barrier()

    pltpu.sync_copy(acc.at[pl.ds(s * RPS, RPS)],
                    out_hbm.at[c, pl.ds(s * RPS, RPS)])


def _sc_agg(y, gidx, dst):
    mesh = plsc.VectorSubcoreMesh(core_axis_name="c", subcore_axis_name="s")
    fn = pl.kernel(
        _sc_body,
        out_type=jax.ShapeDtypeStruct((2, NPAD, FH), f32),
        mesh=mesh,
        scratch_types=[
            pltpu.VMEM((IB, ECH), jnp.int32),
            pltpu.VMEM((IB, ECH), jnp.int32),
            pltpu.VMEM((ZROWS, FH), f32),
            pltpu.VMEM((ECH, FH), f32),
            pltpu.VMEM((ECH, FH), f32),
            pltpu.VMEM_SHARED((NPAD, FH), f32),
            pltpu.SemaphoreType.DMA,
            pltpu.SemaphoreType.DMA,
            pltpu.SemaphoreType.DMA,
            pltpu.SemaphoreType.DMA,
        ],
    )
    return fn(y.reshape(2, K * NPAD, FH), gidx, dst)


def _yspec():
    return pl.BlockSpec((2, K, NB, FH), lambda i: (0, 0, i, 0))


def _aspec():
    return pl.BlockSpec((2, NB, FH), lambda i: (0, i, 0))


def _nspec(w):
    return pl.BlockSpec((NB, w), lambda i: (i, 0))


def _fspec(shape):
    return pl.BlockSpec(shape, lambda i: (0,) * len(shape))


def _cat(a_ref):
    return jnp.concatenate([a_ref[0], a_ref[1]], axis=-1)


def _emit(h8, w_ref, y_ref):
    for k in range(K):
        row = jnp.dot(h8, w_ref[k], preferred_element_type=f32)
        y_ref[0, k] = row[:, :FH]
        y_ref[1, k] = row[:, FH:]


def _t_first(x_ref, w_ref, y_ref):
    x = x_ref[...]
    h8 = jnp.concatenate([x] * OUT, axis=1)
    _emit(h8, w_ref, y_ref)


def _t_lin(a_ref, b0_ref, w_ref, y_ref):
    a = _cat(a_ref)
    h8 = jnp.maximum(a + b0_ref[0], 0.0)
    _emit(h8, w_ref, y_ref)


def _t_res(a1_ref, a3_ref, b0_ref, b2_ref, w_ref, y_ref):
    h = jnp.maximum(_cat(a1_ref) + b0_ref[0], 0.0)
    r2 = _cat(a3_ref) + b2_ref[0]
    h8 = jnp.maximum(h + r2, 0.0)
    _emit(h8, w_ref, y_ref)


def _t_pr(a4_ref, b3_ref, w_ref, y_ref):
    a = _cat(a_ref=a4_ref)
    g = a + b3_ref[0]
    xg = g[:, :CH]
    parts = [xg] + [xg + g[:, oi * CH:(oi + 1) * CH] for oi in range(1, OUT)]
    h8 = jnp.concatenate(parts, axis=1)
    _emit(h8, w_ref, y_ref)


def _t_head(apr_ref, prb_ref, w1_ref, b1_ref, w2_ref, b2_ref, out_ref):
    a = _cat(apr_ref).reshape(NB, OUT, CH)
    cols = []
    for oi in range(OUT):
        t = a[:, oi] + prb_ref[oi]
        y1 = jnp.maximum(
            jnp.dot(t, w1_ref[oi], preferred_element_type=f32) + b1_ref[oi],
            0.0)
        z = jnp.dot(y1, w2_ref[oi], preferred_element_type=f32) + b2_ref[oi]
        cols.append(1.0 / (1.0 + jnp.exp(-z)))
    out_ref[...] = jnp.concatenate(cols, axis=1)


def _call(body, in_arrs, in_specs, out_shape, out_spec):
    return pl.pallas_call(
        body, grid=(GRID,), in_specs=in_specs,
        out_specs=out_spec,
        out_shape=jax.ShapeDtypeStruct(out_shape, f32),
    )(*in_arrs)


def kernel(x_low, x_occ, edge_index, edge_offset,
           bi_W0, bi_b0, bi_W1, bi_b1, bi_W2, bi_b2, bi_W3, bi_b3,
           pr_W, pr_b, ml_W1, ml_b1, ml_W2, ml_b2,
           ot_W0, ot_b0, ot_W1, ot_b1, ot_W2, ot_b2, ot_W3, ot_b3):
    src = edge_index[0]
    dstv = edge_index[1]
    off = edge_offset

    gidx = off * NPAD + src
    pad = EPAD - E
    gidx_t = jnp.concatenate(
        [gidx, jnp.zeros((pad,), jnp.int32)]).reshape(NSUB, NCHUNK, ECH)
    dst_t = jnp.concatenate(
        [dstv, jnp.full((pad,), N, jnp.int32)]).reshape(NSUB, NCHUNK, ECH)

    x8 = jnp.concatenate(
        [jnp.pad(x_low, ((0, NPAD - N), (0, 0))),
         jnp.pad(x_occ[:, :OUT - 1], ((0, NPAD - N), (0, 0)))], axis=1)

    cmask = (jnp.arange(OUT - 1)[None, :] <= jnp.arange(OUT - 1)[:, None])
    W0m = ot_W0 * cmask[:, None, :, None].astype(f32)
    wp1 = jnp.concatenate(
        [jnp.pad(bi_W0[None], ((0, 0), (0, 0), (0, OUT - 1), (0, 0))),
         jnp.pad(W0m, ((0, 0), (0, 0), (1, 0), (0, 0)))], axis=0)

    def blockdiag(w):
        B, Kk, Cb, Co = w.shape
        wt = jnp.moveaxis(w, 1, 0)
        out = jnp.zeros((Kk, B, Cb, B, Co), w.dtype)
        for b in range(B):
            out = out.at[:, b, :, b, :].set(wt[:, b])
        return out.reshape(Kk, B * Cb, B * Co)

    wp1 = blockdiag(wp1)
    w1 = blockdiag(jnp.concatenate([bi_W1[None], ot_W1], axis=0))
    w2 = blockdiag(jnp.concatenate([bi_W2[None], ot_W2], axis=0))
    w3 = blockdiag(jnp.concatenate([bi_W3[None], ot_W3], axis=0))
    wpr = blockdiag(pr_W)
    b0 = jnp.concatenate([bi_b0[None], ot_b0], axis=0).reshape(1, OUT * CH)
    b1 = jnp.concatenate([bi_b1[None], ot_b1], axis=0).reshape(1, OUT * CH)
    b2 = jnp.concatenate([bi_b2[None], ot_b2], axis=0).reshape(1, OUT * CH)
    b3 = jnp.concatenate([bi_b3[None], ot_b3], axis=0).reshape(1, OUT * CH)

    def agg(y):
        return _sc_agg(y, gidx_t, dst_t)

    yshape = (2, K, NPAD, FH)

    y = _call(_t_first, (x8, wp1), [_nspec(OUT), _fspec(wp1.shape)],
              yshape, _yspec())
    a1 = agg(y)
    y = _call(_t_lin, (a1, b0, w1),
              [_aspec(), _fspec(b0.shape), _fspec(w1.shape)],
              yshape, _yspec())
    a2 = agg(y)
    y = _call(_t_lin, (a2, b1, w2),
              [_aspec(), _fspec(b1.shape), _fspec(w2.shape)],
              yshape, _yspec())
    a3 = agg(y)
    y = _call(_t_res, (a1, a3, b0, b2, w3),
              [_aspec(), _aspec(), _fspec(b0.shape), _fspec(b2.shape),
               _fspec(w3.shape)],
              yshape, _yspec())
    a4 = agg(y)
    y = _call(_t_pr, (a4, b3, wpr),
              [_aspec(), _fspec(b3.shape), _fspec(wpr.shape)],
              yshape, _yspec())
    apr = agg(y)

    out = _call(_t_head, (apr, pr_b, ml_W1, ml_b1, ml_W2, ml_b2),
                [_aspec(), _fspec(pr_b.shape), _fspec(ml_W1.shape),
                 _fspec(ml_b1.shape), _fspec(ml_W2.shape),
                 _fspec(ml_b2.shape)],
                (NPAD, OUT), _nspec(OUT))
    return out[:N]
